# baseline (speedup 1.0000x reference)
import functools

import numpy as np
import jax
import jax.numpy as jnp
from jax import lax
from jax.experimental import pallas as pl
from jax.experimental.pallas import tpu as pltpu
from jax.experimental.pallas import tpu_sc as plsc

_NF = 26
_ROWS_PER_FIELD = 100000
_BATCH = 16384
_B = _BATCH * _NF
_D = 32
_NW = 32
_BPW = _B // _NW
_BATCH_PW = _BATCH // _NW
_C = 832
_NCHUNK = _BPW // _C
_VL = 16
_P = 208
_NJ = _P // _VL
_NBLK = _BPW // _P

_k = np.arange(_P, dtype=np.int32)
_CONSTS = np.concatenate([
    (_k % _NF).reshape(_NJ, _VL).ravel(),
    (_k // _NF).reshape(_NJ, _VL).ravel(),
    ((_k % _NF) * _ROWS_PER_FIELD).reshape(_NJ, _VL).ravel(),
])

_mesh = plsc.VectorSubcoreMesh(core_axis_name="c", subcore_axis_name="s")


@functools.partial(
    pl.kernel,
    out_type=jax.ShapeDtypeStruct((_B, _D), jnp.float32),
    mesh=_mesh,
    compiler_params=pltpu.CompilerParams(
        use_tc_tiling_on_sc=False, needs_layout_passes=False
    ),
    scratch_types=[
        pltpu.VMEM((_NF, _BATCH_PW), jnp.int32),
        pltpu.VMEM((3 * _P,), jnp.int32),
        pltpu.VMEM((_BPW,), jnp.int32),
        pltpu.VMEM((_C, _D), jnp.float32),
        pltpu.VMEM((_C, _D), jnp.float32),
        pltpu.SemaphoreType.DMA,
        pltpu.SemaphoreType.DMA,
        pltpu.SemaphoreType.DMA,
        pltpu.SemaphoreType.DMA,
    ],
)
def _embed_gather(xt_hbm, consts_hbm, table_hbm, out_hbm,
                  stage_v, consts_v, idx_v, rows0, rows1,
                  gsem0, gsem1, osem0, osem1):
    wid = lax.axis_index("s") * 2 + lax.axis_index("c")
    base = wid * _BPW

    pltpu.sync_copy(consts_hbm, consts_v)
    pltpu.sync_copy(
        xt_hbm.at[:, pl.ds(wid * _BATCH_PW, _BATCH_PW)], stage_v
    )

    for j in range(_NJ):
        fvec = consts_v[pl.ds(j * _VL, _VL)]
        bvec = consts_v[pl.ds(_P + j * _VL, _VL)]
        ovec = consts_v[pl.ds(2 * _P + j * _VL, _VL)]

        def _blk(blk, carry, fvec=fvec, bvec=bvec, ovec=ovec, j=j):
            vals = plsc.load_gather(
                stage_v, [fvec, bvec + jnp.full((_VL,), 8, jnp.int32) * blk]
            )
            idx_v[pl.ds(blk * _P + j * _VL, _VL)] = vals + ovec
            return carry

        lax.fori_loop(0, _NBLK, _blk, 0)

    bufs = (rows0, rows1)
    gsems = (gsem0, gsem1)
    osems = (osem0, osem1)

    def _start_gather(g):
        return pltpu.async_copy(
            table_hbm.at[idx_v.at[pl.ds(g * _C, _C)]], bufs[g % 2], gsems[g % 2]
        )

    def _start_out(g):
        return pltpu.async_copy(
            bufs[g % 2], out_hbm.at[pl.ds(base + g * _C, _C)], osems[g % 2]
        )

    gcp = [None, None]
    ocp = [None, None]
    gcp[0] = _start_gather(0)
    for g in range(1, _NCHUNK + 1):
        if g < _NCHUNK:
            if ocp[g % 2] is not None:
                ocp[g % 2].wait()
            gcp[g % 2] = _start_gather(g)
        gcp[(g - 1) % 2].wait()
        ocp[(g - 1) % 2] = _start_out(g - 1)
    ocp[(_NCHUNK - 2) % 2].wait()
    ocp[(_NCHUNK - 1) % 2].wait()


def kernel(x, table):
    consts = jnp.asarray(_CONSTS)
    out = _embed_gather(x.T, consts, table)
    return out.reshape(_BATCH, _NF, _D)
